# baseline (speedup 1.0000x reference)
import functools

import jax
import jax.numpy as jnp
from jax import lax
from jax.experimental import pallas as pl
from jax.experimental.pallas import tpu as pltpu
from jax.experimental.pallas import tpu_sc as plsc

N = 10000
D = 128
NP = 10240
NC = 2
NS = 16
NW = NC * NS
K = 128
CB = 8
BLK = 1024

_MESH = plsc.VectorSubcoreMesh(core_axis_name="c", subcore_axis_name="s")


def _sc_degree(dst3, ones_kd, zeros_d):
    ch = dst3.shape[1]

    @functools.partial(
        pl.kernel,
        out_type=jax.ShapeDtypeStruct((NC, NP, D), jnp.float32),
        mesh=_MESH,
        scratch_types=[
            pltpu.VMEM((ch, K), jnp.int32),
            pltpu.VMEM((K, D), jnp.float32),
            pltpu.VMEM_SHARED((NP, D), jnp.float32),
            pltpu.SemaphoreType.DMA,
        ],
    )
    def k(dst_hbm, ones_hbm, z_hbm, out_hbm, dst_v, ones_v, acc_sh, sem0):
        c = lax.axis_index("c")
        s = lax.axis_index("s")
        wid = s * NC + c
        rows = NP // NS
        pltpu.sync_copy(z_hbm.at[pl.ds(s * rows, rows)],
                        acc_sh.at[pl.ds(s * rows, rows)])
        pltpu.sync_copy(dst_hbm.at[wid], dst_v)
        pltpu.sync_copy(ones_hbm, ones_v)
        plsc.subcore_barrier()

        def body(j, carry):
            pltpu.async_copy(ones_v, acc_sh.at[dst_v.at[j]], sem0, add=True)
            return carry

        lax.fori_loop(0, ch, body, 0)

        def drain(j, carry):
            pltpu.make_async_copy(ones_v, acc_sh.at[dst_v.at[0]],
                                  sem0).wait()
            return carry

        lax.fori_loop(0, ch, drain, 0)
        plsc.subcore_barrier()
        pltpu.sync_copy(acc_sh.at[pl.ds(s * rows, rows)],
                        out_hbm.at[c, pl.ds(s * rows, rows)])

    return k(dst3, ones_kd, zeros_d)


def _sc_spread(y, src3, dst3, zeros_d):
    ch = src3.shape[1]
    ngrp = ch // CB

    @functools.partial(
        pl.kernel,
        out_type=jax.ShapeDtypeStruct((NC, NP, D), jnp.float32),
        mesh=_MESH,
        scratch_types=[
            pltpu.VMEM((2, CB, K), jnp.int32),
            pltpu.VMEM((2, CB, K), jnp.int32),
            pltpu.VMEM((K, D), jnp.float32),
            pltpu.VMEM((K, D), jnp.float32),
            pltpu.VMEM_SHARED((NP, D), jnp.float32),
            pltpu.SemaphoreType.DMA,
            pltpu.SemaphoreType.DMA,
            pltpu.SemaphoreType.DMA,
        ],
    )
    def k(y_hbm, src_hbm, dst_hbm, z_hbm, out_hbm,
          src_v, dst_v, rows_a, rows_b, acc_sh, sem_a, sem_b, sem_i):
        c = lax.axis_index("c")
        s = lax.axis_index("s")
        wid = s * NC + c
        rows = NP // NS
        pltpu.sync_copy(z_hbm.at[pl.ds(s * rows, rows)],
                        acc_sh.at[pl.ds(s * rows, rows)])
        pltpu.sync_copy(src_hbm.at[wid, pl.ds(0, CB)], src_v.at[0])
        pltpu.sync_copy(dst_hbm.at[wid, pl.ds(0, CB)], dst_v.at[0])
        plsc.subcore_barrier()

        pltpu.async_copy(y_hbm.at[src_v.at[0, 0]], rows_a, sem_a)
        pltpu.async_copy(y_hbm.at[src_v.at[0, 1]], rows_b, sem_b)

        def grp(g, carry):
            par = g % 2
            nxt = 1 - par
            more = g + 1 < ngrp

            @pl.when(more)
            def _():
                pltpu.async_copy(src_hbm.at[wid, pl.ds((g + 1) * CB, CB)],
                                 src_v.at[nxt], sem_i)
                pltpu.async_copy(dst_hbm.at[wid, pl.ds((g + 1) * CB, CB)],
                                 dst_v.at[nxt], sem_i)

            for t in range(CB // 2):
                ja = 2 * t
                jb = 2 * t + 1
                last = t == CB // 2 - 1

                pltpu.make_async_copy(y_hbm.at[src_v.at[par, ja]],
                                      rows_a, sem_a).wait()
                pltpu.sync_copy(rows_a, acc_sh.at[dst_v.at[par, ja]],
                                add=True)
                if not last:
                    pltpu.async_copy(y_hbm.at[src_v.at[par, ja + 2]],
                                     rows_a, sem_a)
                else:
                    @pl.when(more)
                    def _():
                        pltpu.make_async_copy(
                            src_hbm.at[wid, pl.ds(0, CB)], src_v.at[nxt],
                            sem_i).wait()
                        pltpu.make_async_copy(
                            dst_hbm.at[wid, pl.ds(0, CB)], dst_v.at[nxt],
                            sem_i).wait()
                        pltpu.async_copy(y_hbm.at[src_v.at[nxt, 0]],
                                         rows_a, sem_a)

                pltpu.make_async_copy(y_hbm.at[src_v.at[par, jb]],
                                      rows_b, sem_b).wait()
                pltpu.sync_copy(rows_b, acc_sh.at[dst_v.at[par, jb]],
                                add=True)
                if not last:
                    pltpu.async_copy(y_hbm.at[src_v.at[par, jb + 2]],
                                     rows_b, sem_b)
                else:
                    @pl.when(more)
                    def _():
                        pltpu.async_copy(y_hbm.at[src_v.at[nxt, 1]],
                                         rows_b, sem_b)

            return carry

        lax.fori_loop(0, ngrp, grp, 0)
        plsc.subcore_barrier()
        pltpu.sync_copy(acc_sh.at[pl.ds(s * rows, rows)],
                        out_hbm.at[c, pl.ds(s * rows, rows)])

    return k(y, src3, dst3, zeros_d)


def _tc_xw(xp, w1):
    def body(x_ref, w_ref, o_ref):
        o_ref[...] = jnp.dot(x_ref[...], w_ref[...],
                             preferred_element_type=jnp.float32)

    return pl.pallas_call(
        body,
        grid=(NP // BLK,),
        in_specs=[
            pl.BlockSpec((BLK, D), lambda i: (i, 0)),
            pl.BlockSpec((D, D), lambda i: (0, 0)),
        ],
        out_specs=pl.BlockSpec((BLK, D), lambda i: (i, 0)),
        out_shape=jax.ShapeDtypeStruct((NP, D), jnp.float32),
    )(xp, w1)


def _tc_pre(xw, degacc):
    def body(xw_ref, deg_ref, y_ref, dinv_ref):
        d = deg_ref[0, :, 0:1] + deg_ref[1, :, 0:1] + 1.0
        dinv = lax.rsqrt(d)
        y_ref[...] = xw_ref[...] * dinv
        dinv_ref[...] = dinv

    return pl.pallas_call(
        body,
        grid=(NP // BLK,),
        in_specs=[
            pl.BlockSpec((BLK, D), lambda i: (i, 0)),
            pl.BlockSpec((NC, BLK, D), lambda i: (0, i, 0)),
        ],
        out_specs=[
            pl.BlockSpec((BLK, D), lambda i: (i, 0)),
            pl.BlockSpec((BLK, 1), lambda i: (i, 0)),
        ],
        out_shape=[
            jax.ShapeDtypeStruct((NP, D), jnp.float32),
            jax.ShapeDtypeStruct((NP, 1), jnp.float32),
        ],
    )(xw, degacc)


def _tc_mid(acc, y1, dinv, b1, w2):
    def body(acc_ref, y1_ref, dinv_ref, b_ref, w_ref, y2_ref):
        t = (acc_ref[0] + acc_ref[1] + y1_ref[...]) * dinv_ref[...] + b_ref[...]
        h = jnp.maximum(t, 0.0)
        y2_ref[...] = jnp.dot(h, w_ref[...],
                              preferred_element_type=jnp.float32) * dinv_ref[...]

    return pl.pallas_call(
        body,
        grid=(NP // BLK,),
        in_specs=[
            pl.BlockSpec((NC, BLK, D), lambda i: (0, i, 0)),
            pl.BlockSpec((BLK, D), lambda i: (i, 0)),
            pl.BlockSpec((BLK, 1), lambda i: (i, 0)),
            pl.BlockSpec((1, D), lambda i: (0, 0)),
            pl.BlockSpec((D, D), lambda i: (0, 0)),
        ],
        out_specs=pl.BlockSpec((BLK, D), lambda i: (i, 0)),
        out_shape=jax.ShapeDtypeStruct((NP, D), jnp.float32),
    )(acc, y1, dinv, b1, w2)


def _tc_fin(acc, y2, dinv, b2):
    def body(acc_ref, y2_ref, dinv_ref, b_ref, o_ref):
        o_ref[...] = ((acc_ref[0] + acc_ref[1] + y2_ref[...])
                      * dinv_ref[...] + b_ref[...])

    return pl.pallas_call(
        body,
        grid=(NP // BLK,),
        in_specs=[
            pl.BlockSpec((NC, BLK, D), lambda i: (0, i, 0)),
            pl.BlockSpec((BLK, D), lambda i: (i, 0)),
            pl.BlockSpec((BLK, 1), lambda i: (i, 0)),
            pl.BlockSpec((1, D), lambda i: (0, 0)),
        ],
        out_specs=pl.BlockSpec((BLK, D), lambda i: (i, 0)),
        out_shape=jax.ShapeDtypeStruct((NP, D), jnp.float32),
    )(acc, y2, dinv, b2)


def kernel(x, edge_index, W1, b1, W2, b2):
    e = edge_index.shape[1]
    ch = -(-e // (NW * K))
    ch = -(-ch // CB) * CB
    ep = ch * NW * K
    ei = edge_index.astype(jnp.int32)
    pad = N + jnp.arange(ep - e, dtype=jnp.int32) % (NP - N)
    src3 = jnp.concatenate([ei[0], pad]).reshape(NW, ch, K)
    dst3 = jnp.concatenate([ei[1], pad]).reshape(NW, ch, K)
    xp = jnp.pad(x.astype(jnp.float32), ((0, NP - N), (0, 0)))
    zeros_d = jnp.zeros((NP, D), jnp.float32)

    ones_kd = jnp.ones((K, D), jnp.float32)
    xw = _tc_xw(xp, W1.astype(jnp.float32))
    degacc = _sc_degree(dst3, ones_kd, zeros_d)
    y1, dinv = _tc_pre(xw, degacc)
    acc1 = _sc_spread(y1, src3, dst3, zeros_d)
    y2 = _tc_mid(acc1, y1, dinv, b1.reshape(1, D).astype(jnp.float32),
                 W2.astype(jnp.float32))
    acc2 = _sc_spread(y2, src3, dst3, zeros_d)
    out = _tc_fin(acc2, y2, dinv, b2.reshape(1, D).astype(jnp.float32))
    return out[:N]


# speedup vs baseline: 1.0044x; 1.0044x over previous
import functools

import jax
import jax.numpy as jnp
from jax import lax
from jax.experimental import pallas as pl
from jax.experimental.pallas import tpu as pltpu
from jax.experimental.pallas import tpu_sc as plsc

N = 10000
D = 128
NP = 10240
NC = 2
NS = 16
NW = NC * NS
K = 128
CB = 8
BLK = 1024

_MESH = plsc.VectorSubcoreMesh(core_axis_name="c", subcore_axis_name="s")


def _sc_degree(dst3, ones_kd, zeros_d):
    ch = dst3.shape[1]

    @functools.partial(
        pl.kernel,
        out_type=jax.ShapeDtypeStruct((NC, NP, D), jnp.float32),
        mesh=_MESH,
        scratch_types=[
            pltpu.VMEM((ch, K), jnp.int32),
            pltpu.VMEM((K, D), jnp.float32),
            pltpu.VMEM_SHARED((NP, D), jnp.float32),
        ],
    )
    def k(dst_hbm, ones_hbm, z_hbm, out_hbm, dst_v, ones_v, acc_sh):
        c = lax.axis_index("c")
        s = lax.axis_index("s")
        wid = s * NC + c
        rows = NP // NS
        pltpu.sync_copy(z_hbm.at[pl.ds(s * rows, rows)],
                        acc_sh.at[pl.ds(s * rows, rows)])
        pltpu.sync_copy(dst_hbm.at[wid], dst_v)
        pltpu.sync_copy(ones_hbm, ones_v)
        plsc.subcore_barrier()

        def body(j, carry):
            pltpu.sync_copy(ones_v, acc_sh.at[dst_v.at[j]], add=True)
            return carry

        lax.fori_loop(0, ch, body, 0)
        plsc.subcore_barrier()
        pltpu.sync_copy(acc_sh.at[pl.ds(s * rows, rows)],
                        out_hbm.at[c, pl.ds(s * rows, rows)])

    return k(dst3, ones_kd, zeros_d)


def _sc_spread(y, src3, dst3, zeros_d):
    ch = src3.shape[1]
    ngrp = ch // CB

    @functools.partial(
        pl.kernel,
        out_type=jax.ShapeDtypeStruct((NC, NP, D), jnp.float32),
        mesh=_MESH,
        scratch_types=[
            pltpu.VMEM((2, CB, K), jnp.int32),
            pltpu.VMEM((2, CB, K), jnp.int32),
            pltpu.VMEM((K, D), jnp.float32),
            pltpu.VMEM((K, D), jnp.float32),
            pltpu.VMEM_SHARED((NP, D), jnp.float32),
            pltpu.SemaphoreType.DMA,
            pltpu.SemaphoreType.DMA,
            pltpu.SemaphoreType.DMA,
        ],
    )
    def k(y_hbm, src_hbm, dst_hbm, z_hbm, out_hbm,
          src_v, dst_v, rows_a, rows_b, acc_sh, sem_a, sem_b, sem_i):
        c = lax.axis_index("c")
        s = lax.axis_index("s")
        wid = s * NC + c
        rows = NP // NS
        pltpu.sync_copy(z_hbm.at[pl.ds(s * rows, rows)],
                        acc_sh.at[pl.ds(s * rows, rows)])
        pltpu.sync_copy(src_hbm.at[wid, pl.ds(0, CB)], src_v.at[0])
        pltpu.sync_copy(dst_hbm.at[wid, pl.ds(0, CB)], dst_v.at[0])
        plsc.subcore_barrier()

        pltpu.async_copy(y_hbm.at[src_v.at[0, 0]], rows_a, sem_a)
        pltpu.async_copy(y_hbm.at[src_v.at[0, 1]], rows_b, sem_b)

        def grp(g, carry):
            par = g % 2
            nxt = 1 - par
            more = g + 1 < ngrp

            @pl.when(more)
            def _():
                pltpu.async_copy(src_hbm.at[wid, pl.ds((g + 1) * CB, CB)],
                                 src_v.at[nxt], sem_i)
                pltpu.async_copy(dst_hbm.at[wid, pl.ds((g + 1) * CB, CB)],
                                 dst_v.at[nxt], sem_i)

            for t in range(CB // 2):
                ja = 2 * t
                jb = 2 * t + 1
                last = t == CB // 2 - 1

                pltpu.make_async_copy(y_hbm.at[src_v.at[par, ja]],
                                      rows_a, sem_a).wait()
                pltpu.sync_copy(rows_a, acc_sh.at[dst_v.at[par, ja]],
                                add=True)
                if not last:
                    pltpu.async_copy(y_hbm.at[src_v.at[par, ja + 2]],
                                     rows_a, sem_a)
                else:
                    @pl.when(more)
                    def _():
                        pltpu.make_async_copy(
                            src_hbm.at[wid, pl.ds(0, CB)], src_v.at[nxt],
                            sem_i).wait()
                        pltpu.make_async_copy(
                            dst_hbm.at[wid, pl.ds(0, CB)], dst_v.at[nxt],
                            sem_i).wait()
                        pltpu.async_copy(y_hbm.at[src_v.at[nxt, 0]],
                                         rows_a, sem_a)

                pltpu.make_async_copy(y_hbm.at[src_v.at[par, jb]],
                                      rows_b, sem_b).wait()
                pltpu.sync_copy(rows_b, acc_sh.at[dst_v.at[par, jb]],
                                add=True)
                if not last:
                    pltpu.async_copy(y_hbm.at[src_v.at[par, jb + 2]],
                                     rows_b, sem_b)
                else:
                    @pl.when(more)
                    def _():
                        pltpu.async_copy(y_hbm.at[src_v.at[nxt, 1]],
                                         rows_b, sem_b)

            return carry

        lax.fori_loop(0, ngrp, grp, 0)
        plsc.subcore_barrier()
        pltpu.sync_copy(acc_sh.at[pl.ds(s * rows, rows)],
                        out_hbm.at[c, pl.ds(s * rows, rows)])

    return k(y, src3, dst3, zeros_d)


def _tc_pre(xp, w1, degacc):
    def body(x_ref, w_ref, deg_ref, y_ref, dinv_ref):
        d = deg_ref[0, :, 0:1] + deg_ref[1, :, 0:1] + 1.0
        dinv = lax.rsqrt(d)
        y_ref[...] = jnp.dot(x_ref[...], w_ref[...],
                             preferred_element_type=jnp.float32) * dinv
        dinv_ref[...] = dinv

    return pl.pallas_call(
        body,
        grid=(NP // BLK,),
        in_specs=[
            pl.BlockSpec((BLK, D), lambda i: (i, 0)),
            pl.BlockSpec((D, D), lambda i: (0, 0)),
            pl.BlockSpec((NC, BLK, D), lambda i: (0, i, 0)),
        ],
        out_specs=[
            pl.BlockSpec((BLK, D), lambda i: (i, 0)),
            pl.BlockSpec((BLK, 1), lambda i: (i, 0)),
        ],
        out_shape=[
            jax.ShapeDtypeStruct((NP, D), jnp.float32),
            jax.ShapeDtypeStruct((NP, 1), jnp.float32),
        ],
    )(xp, w1, degacc)


def _tc_mid(acc, y1, dinv, b1, w2):
    def body(acc_ref, y1_ref, dinv_ref, b_ref, w_ref, y2_ref):
        t = (acc_ref[0] + acc_ref[1] + y1_ref[...]) * dinv_ref[...] + b_ref[...]
        h = jnp.maximum(t, 0.0)
        y2_ref[...] = jnp.dot(h, w_ref[...],
                              preferred_element_type=jnp.float32) * dinv_ref[...]

    return pl.pallas_call(
        body,
        grid=(NP // BLK,),
        in_specs=[
            pl.BlockSpec((NC, BLK, D), lambda i: (0, i, 0)),
            pl.BlockSpec((BLK, D), lambda i: (i, 0)),
            pl.BlockSpec((BLK, 1), lambda i: (i, 0)),
            pl.BlockSpec((1, D), lambda i: (0, 0)),
            pl.BlockSpec((D, D), lambda i: (0, 0)),
        ],
        out_specs=pl.BlockSpec((BLK, D), lambda i: (i, 0)),
        out_shape=jax.ShapeDtypeStruct((NP, D), jnp.float32),
    )(acc, y1, dinv, b1, w2)


def _tc_fin(acc, y2, dinv, b2):
    def body(acc_ref, y2_ref, dinv_ref, b_ref, o_ref):
        o_ref[...] = ((acc_ref[0] + acc_ref[1] + y2_ref[...])
                      * dinv_ref[...] + b_ref[...])

    return pl.pallas_call(
        body,
        grid=(NP // BLK,),
        in_specs=[
            pl.BlockSpec((NC, BLK, D), lambda i: (0, i, 0)),
            pl.BlockSpec((BLK, D), lambda i: (i, 0)),
            pl.BlockSpec((BLK, 1), lambda i: (i, 0)),
            pl.BlockSpec((1, D), lambda i: (0, 0)),
        ],
        out_specs=pl.BlockSpec((BLK, D), lambda i: (i, 0)),
        out_shape=jax.ShapeDtypeStruct((NP, D), jnp.float32),
    )(acc, y2, dinv, b2)


def kernel(x, edge_index, W1, b1, W2, b2):
    e = edge_index.shape[1]
    ch = -(-e // (NW * K))
    ch = -(-ch // CB) * CB
    ep = ch * NW * K
    ei = edge_index.astype(jnp.int32)
    pad = N + jnp.arange(ep - e, dtype=jnp.int32) % (NP - N)
    src3 = jnp.concatenate([ei[0], pad]).reshape(NW, ch, K)
    dst3 = jnp.concatenate([ei[1], pad]).reshape(NW, ch, K)
    xp = jnp.pad(x.astype(jnp.float32), ((0, NP - N), (0, 0)))
    zeros_d = jnp.zeros((NP, D), jnp.float32)

    ones_kd = jnp.ones((K, D), jnp.float32)
    degacc = _sc_degree(dst3, ones_kd, zeros_d)
    y1, dinv = _tc_pre(xp, W1.astype(jnp.float32), degacc)
    acc1 = _sc_spread(y1, src3, dst3, zeros_d)
    y2 = _tc_mid(acc1, y1, dinv, b1.reshape(1, D).astype(jnp.float32),
                 W2.astype(jnp.float32))
    acc2 = _sc_spread(y2, src3, dst3, zeros_d)
    out = _tc_fin(acc2, y2, dinv, b2.reshape(1, D).astype(jnp.float32))
    return out[:N]
